# baseline (speedup 1.0000x reference)
import jax
import jax.numpy as jnp
from jax import lax
from jax.experimental import pallas as pl
from jax.experimental.pallas import tpu as pltpu
from jax.experimental.pallas import tpu_sc as plsc

VOCAB = 1000000
DIM = 64
BATCH = 16384
NEG = 20

NC = 2
NS = 16
LANES = 16
NPART = DIM // LANES
NW = NC * NS
B_PER_W = BATCH // NW
CB = 32
NCHUNK = B_PER_W // CB
NEG_STREAMS = CB * NEG // 128
PACK = 32
FILL = PACK - (NEG + 1)

TBLK = 512
TGRID = 977
SPLIT = TGRID * TBLK
PROW = DIM + DIM


def _pair_body(xa_ref, xb_ref, o_ref):
    o_ref[...] = jnp.concatenate(
        [jnp.transpose(xa_ref[...]), jnp.transpose(xb_ref[...])], axis=1)


def _to_pairs(table):
    xt = jnp.transpose(table)
    return pl.pallas_call(
        _pair_body,
        grid=(TGRID,),
        in_specs=[pl.BlockSpec((DIM, TBLK), lambda i: (0, i)),
                  pl.BlockSpec((DIM, TBLK), lambda i: (0, i + TGRID))],
        out_specs=pl.BlockSpec((TBLK, PROW), lambda i: (i, 0)),
        out_shape=jax.ShapeDtypeStruct((SPLIT, PROW), jnp.float32),
    )(xt, xt)


def _hsum(acc):
    for sh in (8, 4, 2, 1):
        perm = lax.iota(jnp.int32, LANES) ^ sh
        acc = acc + acc.at[perm].get(mode="promise_in_bounds")
    return acc


def _sc_body(tgt_hbm, ctx_hbm, neg_hbm, in_pairs, out_pairs, scores_out,
             tidx, cidx, nidx, ptidx, pcidx, pnidx,
             t_rows, c_rows, n_rows, score_buf, sem):
    wid = lax.axis_index("s") * NC + lax.axis_index("c")

    def chunk_body(ci, _):
        gbase = wid * B_PER_W + ci * CB

        pltpu.sync_copy(tgt_hbm.at[pl.ds(gbase, CB)], tidx.at[pl.ds(0, CB)])
        pltpu.sync_copy(ctx_hbm.at[pl.ds(gbase, CB)], cidx.at[pl.ds(0, CB)])
        for j in range(NEG_STREAMS):
            pltpu.sync_copy(neg_hbm.at[pl.ds(gbase * NEG + j * 128, 128)],
                            nidx.at[pl.ds(j * 128, 128)])

        for l in range(CB // LANES):
            v = tidx[pl.ds(l * LANES, LANES)]
            ptidx[pl.ds(l * LANES, LANES)] = jnp.where(v >= SPLIT, v - SPLIT, v)
            v = cidx[pl.ds(l * LANES, LANES)]
            pcidx[pl.ds(l * LANES, LANES)] = jnp.where(v >= SPLIT, v - SPLIT, v)
        for j in range(NEG_STREAMS):
            for l in range(128 // LANES):
                v = nidx[pl.ds(j * 128 + l * LANES, LANES)]
                pnidx[j, pl.ds(l * LANES, LANES)] = (
                    jnp.where(v >= SPLIT, v - SPLIT, v))

        copies = [pltpu.async_copy(in_pairs.at[ptidx], t_rows, sem),
                  pltpu.async_copy(out_pairs.at[pcidx], c_rows, sem)]
        for j in range(NEG_STREAMS):
            copies.append(pltpu.async_copy(
                out_pairs.at[pnidx.at[j]], n_rows.at[pl.ds(j * 128, 128)], sem))
        for c in copies:
            c.wait()

        def elem_body(b, _):
            toff = jnp.where(tidx[pl.ds(b, LANES)][0] >= SPLIT, DIM, 0)
            coff = jnp.where(cidx[pl.ds(b, LANES)][0] >= SPLIT, DIM, 0)
            t = [t_rows[b, pl.ds(toff + i * LANES, LANES)]
                 for i in range(NPART)]
            tn = [-x for x in t]
            c = [c_rows[b, pl.ds(coff + i * LANES, LANES)]
                 for i in range(NPART)]

            def dot(a_parts, b_parts):
                acc = a_parts[0] * b_parts[0]
                for i in range(1, NPART):
                    acc = acc + a_parts[i] * b_parts[i]
                return _hsum(acc)

            def neg_dot(k):
                r = b * NEG + k
                nv = nidx[pl.ds(r, LANES)][0]
                noff = jnp.where(nv >= SPLIT, DIM, 0)
                n = [n_rows[r, pl.ds(noff + i * LANES, LANES)]
                     for i in range(NPART)]
                return dot(tn, n)

            pack_a = dot(t, c)
            for k in range(15):
                mask = lax.iota(jnp.int32, LANES) == (k + 1)
                pack_a = jnp.where(mask, neg_dot(k), pack_a)
            pack_b = jnp.zeros((LANES,), jnp.float32)
            for k in range(15, NEG):
                mask = lax.iota(jnp.int32, LANES) == (k - 15)
                pack_b = jnp.where(mask, neg_dot(k), pack_b)

            score_buf[pl.ds(b * PACK, LANES)] = pack_a
            score_buf[pl.ds(b * PACK + LANES, LANES)] = pack_b
            return ()

        lax.fori_loop(0, CB, elem_body, (), unroll=False)

        pltpu.sync_copy(score_buf, scores_out.at[pl.ds(gbase * PACK, CB * PACK)])
        return ()

    lax.fori_loop(0, NCHUNK, chunk_body, (), unroll=False)


def _scores_sc(tgt, ctx, negs, in_pairs, out_pairs):
    mesh = plsc.VectorSubcoreMesh(core_axis_name="c", subcore_axis_name="s")
    f = pl.kernel(
        _sc_body,
        out_type=jax.ShapeDtypeStruct((BATCH * PACK,), jnp.float32),
        mesh=mesh,
        scratch_types=[
            pltpu.VMEM((CB + LANES,), jnp.int32),
            pltpu.VMEM((CB + LANES,), jnp.int32),
            pltpu.VMEM((NEG_STREAMS * 128 + LANES,), jnp.int32),
            pltpu.VMEM((CB,), jnp.int32),
            pltpu.VMEM((CB,), jnp.int32),
            pltpu.VMEM((NEG_STREAMS, 128), jnp.int32),
            pltpu.VMEM((CB, PROW), jnp.float32),
            pltpu.VMEM((CB, PROW), jnp.float32),
            pltpu.VMEM((CB * NEG, PROW), jnp.float32),
            pltpu.VMEM((CB * PACK,), jnp.float32),
            pltpu.SemaphoreType.DMA,
        ],
        compiler_params=pltpu.CompilerParams(use_tc_tiling_on_sc=True),
    )
    return f(tgt, ctx, negs, in_pairs, out_pairs)


def _loss_body(y_ref, out_ref):
    total = jnp.sum(jax.nn.log_sigmoid(y_ref[...]))
    valid = total + FILL * BATCH * jnp.float32(jnp.log(2.0))
    out_ref[0, 0] = -valid / BATCH


def _loss_tc(scores):
    out = pl.pallas_call(
        _loss_body,
        out_shape=jax.ShapeDtypeStruct((1, 1), jnp.float32),
        in_specs=[pl.BlockSpec(memory_space=pltpu.VMEM)],
        out_specs=pl.BlockSpec(memory_space=pltpu.SMEM),
    )(scores.reshape(BATCH * PACK // 128, 128))
    return out[0, 0]


@jax.jit
def kernel(target_word, context_word, negative_words,
           input_embeddings, output_embeddings):
    tgt = target_word.astype(jnp.int32)
    ctx = context_word.astype(jnp.int32)
    negs = negative_words.astype(jnp.int32).reshape(BATCH * NEG)
    in_pairs = _to_pairs(input_embeddings)
    out_pairs = _to_pairs(output_embeddings)
    scores = _scores_sc(tgt, ctx, negs, in_pairs, out_pairs)
    return _loss_tc(scores)


# speedup vs baseline: 1.3215x; 1.3215x over previous
import jax
import jax.numpy as jnp
from jax import lax
from jax.experimental import pallas as pl
from jax.experimental.pallas import tpu as pltpu
from jax.experimental.pallas import tpu_sc as plsc

VOCAB = 1000000
DIM = 64
BATCH = 16384
NEG = 20

NC = 2
NS = 16
LANES = 16
NPART = DIM // LANES
NW = NC * NS
B_PER_W = BATCH // NW
CB = 32
NCHUNK = B_PER_W // CB
NEG_STREAMS = CB * NEG // 128
PACK = 32
FILL = PACK - (NEG + 1)
PROW = 2 * DIM


def _hsum(acc):
    for sh in (8, 4, 2, 1):
        perm = lax.iota(jnp.int32, LANES) ^ sh
        acc = acc + acc.at[perm].get(mode="promise_in_bounds")
    return acc


def _sc_body(tgt_hbm, ctx_hbm, neg_hbm, in_emb, out_emb, scores_out,
             tidx, cidx, nidx, t_rows, c_rows, n_rows, score_buf, sem):
    wid = lax.axis_index("s") * NC + lax.axis_index("c")

    def chunk_body(ci, _):
        gbase = wid * B_PER_W + ci * CB

        pltpu.sync_copy(tgt_hbm.at[pl.ds(gbase, CB)], tidx)
        pltpu.sync_copy(ctx_hbm.at[pl.ds(gbase, CB)], cidx)
        for j in range(NEG_STREAMS):
            pltpu.sync_copy(neg_hbm.at[pl.ds(gbase * NEG + j * 128, 128)],
                            nidx.at[j])

        copies = [pltpu.async_copy(in_emb.at[tidx], t_rows, sem),
                  pltpu.async_copy(out_emb.at[cidx], c_rows, sem)]
        for j in range(NEG_STREAMS):
            copies.append(pltpu.async_copy(
                out_emb.at[nidx.at[j]], n_rows.at[pl.ds(j * 128, 128)], sem))
        for c in copies:
            c.wait()

        def elem_body(b, _):
            t = [t_rows[b, pl.ds(i * LANES, LANES)] for i in range(NPART)]
            tn = [-x for x in t]
            c = [c_rows[b, pl.ds(i * LANES, LANES)] for i in range(NPART)]

            def dot(a_parts, b_parts):
                acc = a_parts[0] * b_parts[0]
                for i in range(1, NPART):
                    acc = acc + a_parts[i] * b_parts[i]
                return _hsum(acc)

            def neg_dot(k):
                n = [n_rows[b * NEG + k, pl.ds(i * LANES, LANES)]
                     for i in range(NPART)]
                return dot(tn, n)

            pack_a = dot(t, c)
            for k in range(15):
                mask = lax.iota(jnp.int32, LANES) == (k + 1)
                pack_a = jnp.where(mask, neg_dot(k), pack_a)
            pack_b = jnp.zeros((LANES,), jnp.float32)
            for k in range(15, NEG):
                mask = lax.iota(jnp.int32, LANES) == (k - 15)
                pack_b = jnp.where(mask, neg_dot(k), pack_b)

            score_buf[pl.ds(b * PACK, LANES)] = pack_a
            score_buf[pl.ds(b * PACK + LANES, LANES)] = pack_b
            return ()

        lax.fori_loop(0, CB, elem_body, (), unroll=False)

        pltpu.sync_copy(score_buf, scores_out.at[pl.ds(gbase * PACK, CB * PACK)])
        return ()

    lax.fori_loop(0, NCHUNK, chunk_body, (), unroll=False)


def _scores_sc(tgt, ctx, negs, in_emb, out_emb):
    mesh = plsc.VectorSubcoreMesh(core_axis_name="c", subcore_axis_name="s")
    f = pl.kernel(
        _sc_body,
        out_type=jax.ShapeDtypeStruct((BATCH * PACK,), jnp.float32),
        mesh=mesh,
        scratch_types=[
            pltpu.VMEM((CB,), jnp.int32),
            pltpu.VMEM((CB,), jnp.int32),
            pltpu.VMEM((NEG_STREAMS, 128), jnp.int32),
            pltpu.VMEM((CB, PROW), jnp.float32),
            pltpu.VMEM((CB, PROW), jnp.float32),
            pltpu.VMEM((CB * NEG, PROW), jnp.float32),
            pltpu.VMEM((CB * PACK,), jnp.float32),
            pltpu.SemaphoreType.DMA,
        ],
        compiler_params=pltpu.CompilerParams(use_tc_tiling_on_sc=True),
    )
    return f(tgt, ctx, negs, in_emb, out_emb)


def _loss_body(y_ref, out_ref):
    total = jnp.sum(jax.nn.log_sigmoid(y_ref[...]))
    valid = total + FILL * BATCH * jnp.float32(jnp.log(2.0))
    out_ref[0, 0] = -valid / BATCH


def _loss_tc(scores):
    out = pl.pallas_call(
        _loss_body,
        out_shape=jax.ShapeDtypeStruct((1, 1), jnp.float32),
        in_specs=[pl.BlockSpec(memory_space=pltpu.VMEM)],
        out_specs=pl.BlockSpec(memory_space=pltpu.SMEM),
    )(scores.reshape(BATCH * PACK // 128, 128))
    return out[0, 0]


@jax.jit
def kernel(target_word, context_word, negative_words,
           input_embeddings, output_embeddings):
    tgt = target_word.astype(jnp.int32)
    ctx = context_word.astype(jnp.int32)
    negs = negative_words.astype(jnp.int32).reshape(BATCH * NEG)
    in_p = jnp.pad(input_embeddings, ((0, 0), (0, PROW - DIM)))
    out_p = jnp.pad(output_embeddings, ((0, 0), (0, PROW - DIM)))
    scores = _scores_sc(tgt, ctx, negs, in_p, out_p)
    return _loss_tc(scores)
